# Initial kernel scaffold; baseline (speedup 1.0000x reference)
import jax
import jax.numpy as jnp
from jax.experimental import pallas as pl


def kernel(features, edge_index, W_sg, b_sg, W_head, b_head):
    raise NotImplementedError("write your pallas kernel here")


# speedup vs baseline: 3.9230x; 3.9230x over previous
import functools

import jax
import jax.numpy as jnp
from jax import lax
from jax.experimental import pallas as pl
from jax.experimental.pallas import tpu as pltpu
from jax.experimental.pallas import tpu_sc as plsc

N = 10000
NP = 10240
E = 160000
F = 256
FH = 128
NC = 2
NS = 16
RPT = NP // NS
CH = 128
NCHUNK = E // CH
NCHUNK_H = NCHUNK // 2

_MESH = plsc.VectorSubcoreMesh(core_axis_name="c", subcore_axis_name="s")


def _deg_body(edges, degp, degacc, dstb, onesb):
    c = lax.axis_index("c")
    t = lax.axis_index("s")

    def fill_z(r, carry):
        for j in range(FH // 16):
            onesb[r, pl.ds(j * 16, 16)] = jnp.zeros((16,), jnp.float32)
        return carry

    def fill_1(r, carry):
        for j in range(FH // 16):
            onesb[r, pl.ds(j * 16, 16)] = jnp.ones((16,), jnp.float32)
        return carry

    lax.fori_loop(0, CH, fill_z, 0)

    def zc(k, carry):
        pltpu.sync_copy(onesb, degacc.at[pl.ds(t * RPT + k * CH, CH)])
        return carry

    lax.fori_loop(0, RPT // CH, zc, 0)
    lax.fori_loop(0, CH, fill_1, 0)
    plsc.subcore_barrier()

    n0 = (t * NCHUNK_H) // NS
    n1 = ((t + 1) * NCHUNK_H) // NS

    def chunk(n, carry):
        e0 = c * (E // 2) + n * CH
        pltpu.sync_copy(edges.at[1, pl.ds(e0, CH)], dstb)
        pltpu.sync_copy(onesb, degacc.at[dstb], add=True)
        return carry

    lax.fori_loop(n0, n1, chunk, 0)
    plsc.subcore_barrier()
    pltpu.sync_copy(degacc.at[pl.ds(t * RPT, RPT)],
                    degp.at[pl.ds(c * NP + t * RPT, RPT)])


_DEG_SCRATCH = [
    pltpu.VMEM_SHARED((NP, FH), jnp.float32),
    pltpu.VMEM((CH,), jnp.int32),
    pltpu.VMEM((CH, FH), jnp.float32),
]
_deg_call = pl.kernel(
    _deg_body,
    out_type=jax.ShapeDtypeStruct((NC * NP, FH), jnp.float32),
    mesh=_MESH,
    scratch_types=_DEG_SCRATCH,
)


def _hop_body(gtab, edges, out, acc, adjb, dstb, rows):
    c = lax.axis_index("c")
    t = lax.axis_index("s")

    def zr(r, carry):
        for j in range(FH // 16):
            rows[r, pl.ds(j * 16, 16)] = jnp.zeros((16,), jnp.float32)
        return carry

    lax.fori_loop(0, CH, zr, 0)

    def zc(k, carry):
        pltpu.sync_copy(rows, acc.at[pl.ds(t * RPT + k * CH, CH)])
        return carry

    lax.fori_loop(0, RPT // CH, zc, 0)
    plsc.subcore_barrier()

    cN = c * NP
    n0 = (t * NCHUNK) // NS
    n1 = ((t + 1) * NCHUNK) // NS

    def chunk(n, carry):
        e0 = n * CH
        pltpu.sync_copy(edges.at[0, pl.ds(e0, CH)], adjb)
        pltpu.sync_copy(edges.at[1, pl.ds(e0, CH)], dstb)
        for j in range(CH // 16):
            sl = pl.ds(j * 16, 16)
            adjb[sl] = adjb[sl] + cN
        pltpu.sync_copy(gtab.at[adjb], rows)
        pltpu.sync_copy(rows, acc.at[dstb], add=True)
        return carry

    lax.fori_loop(n0, n1, chunk, 0)
    plsc.subcore_barrier()
    pltpu.sync_copy(acc.at[pl.ds(t * RPT, RPT)],
                    out.at[pl.ds(cN + t * RPT, RPT)])


_HOP_SCRATCH = [
    pltpu.VMEM_SHARED((NP, FH), jnp.float32),
    pltpu.VMEM((CH,), jnp.int32),
    pltpu.VMEM((CH,), jnp.int32),
    pltpu.VMEM((CH, FH), jnp.float32),
]
_hop_call = pl.kernel(
    _hop_body,
    out_type=jax.ShapeDtypeStruct((NC * NP, FH), jnp.float32),
    mesh=_MESH,
    scratch_types=_HOP_SCRATCH,
)


def _prep_body(x_ref, degp_ref, nrm_ref, nrm2_ref, g0_ref):
    d = jnp.maximum(degp_ref[0][:, :16] + degp_ref[1][:, :16], 1.0)
    nr = lax.rsqrt(d)
    nrm_ref[...] = nr
    nrm2_ref[...] = nr * nr
    s = nr[:, :1]
    g0_ref[0] = x_ref[:, :FH] * s
    g0_ref[1] = x_ref[:, FH:] * s


def _prep_call(x, degp):
    bn = 640
    return pl.pallas_call(
        _prep_body,
        grid=(NP // bn,),
        in_specs=[
            pl.BlockSpec((bn, F), lambda i: (i, 0)),
            pl.BlockSpec((NC, bn, FH), lambda i: (0, i, 0)),
        ],
        out_specs=[
            pl.BlockSpec((bn, 16), lambda i: (i, 0)),
            pl.BlockSpec((bn, 16), lambda i: (i, 0)),
            pl.BlockSpec((NC, bn, FH), lambda i: (0, i, 0)),
        ],
        out_shape=[
            jax.ShapeDtypeStruct((NP, 16), jnp.float32),
            jax.ShapeDtypeStruct((NP, 16), jnp.float32),
            jax.ShapeDtypeStruct((NC, NP, FH), jnp.float32),
        ],
    )(x, degp)


def _scale_body(s_ref, nrm2_ref, g_ref):
    s2 = nrm2_ref[:, :1]
    g_ref[0] = s_ref[0] * s2
    g_ref[1] = s_ref[1] * s2


def _scale_call(s, nrm2):
    bn = 640
    return pl.pallas_call(
        _scale_body,
        grid=(NP // bn,),
        in_specs=[
            pl.BlockSpec((NC, bn, FH), lambda i: (0, i, 0)),
            pl.BlockSpec((bn, 16), lambda i: (i, 0)),
        ],
        out_specs=pl.BlockSpec((NC, bn, FH), lambda i: (0, i, 0)),
        out_shape=jax.ShapeDtypeStruct((NC, NP, FH), jnp.float32),
    )(s, nrm2)


def _head_body(s_ref, nrm_ref, wsg_ref, bsg_ref, wh_ref, bh_ref,
               h_ref, log_ref):
    nr = nrm_ref[:, :1]
    p0 = s_ref[0] * nr
    p1 = s_ref[1] * nr
    h = jnp.dot(p0, wsg_ref[:FH, :], preferred_element_type=jnp.float32)
    h = h + jnp.dot(p1, wsg_ref[FH:, :], preferred_element_type=jnp.float32)
    h = h + bsg_ref[...]
    h_ref[...] = h
    log_ref[...] = (
        jnp.dot(h, wh_ref[...], preferred_element_type=jnp.float32)
        + bh_ref[...]
    )


def _head_call(s, nrm, W_sg, b_sg, W_head, b_head):
    bn = 1000
    NH = W_sg.shape[1]
    NCL = W_head.shape[1]
    return pl.pallas_call(
        _head_body,
        grid=(N // bn,),
        in_specs=[
            pl.BlockSpec((NC, bn, FH), lambda i: (0, i, 0)),
            pl.BlockSpec((bn, 16), lambda i: (i, 0)),
            pl.BlockSpec((F, NH), lambda i: (0, 0)),
            pl.BlockSpec((1, NH), lambda i: (0, 0)),
            pl.BlockSpec((NH, NCL), lambda i: (0, 0)),
            pl.BlockSpec((1, NCL), lambda i: (0, 0)),
        ],
        out_specs=[
            pl.BlockSpec((bn, NH), lambda i: (i, 0)),
            pl.BlockSpec((bn, NCL), lambda i: (i, 0)),
        ],
        out_shape=[
            jax.ShapeDtypeStruct((N, NH), jnp.float32),
            jax.ShapeDtypeStruct((N, NCL), jnp.float32),
        ],
    )(s, nrm, W_sg, b_sg, W_head, b_head)


def kernel(features, edge_index, W_sg, b_sg, W_head, b_head):
    degp = _deg_call(edge_index).reshape(NC, NP, FH)
    nrm, nrm2, g0 = _prep_call(features, degp)
    s1 = _hop_call(g0.reshape(NC * NP, FH), edge_index)
    g1 = _scale_call(s1.reshape(NC, NP, FH), nrm2)
    s2 = _hop_call(g1.reshape(NC * NP, FH), edge_index)
    h, logits = _head_call(s2.reshape(NC, NP, FH), nrm,
                           W_sg, b_sg.reshape(1, -1),
                           W_head, b_head.reshape(1, -1))
    return (h, logits)


# speedup vs baseline: 6.2635x; 1.5966x over previous
import functools

import jax
import jax.numpy as jnp
from jax import lax
from jax.experimental import pallas as pl
from jax.experimental.pallas import tpu as pltpu
from jax.experimental.pallas import tpu_sc as plsc

N = 10000
NP = 10240
E = 160000
F = 256
FH = 128
NC = 2
NS = 16
RPT = NP // NS
CH = 128
NCHUNK = E // CH
NCHUNK_H = NCHUNK // 2

_MESH = plsc.VectorSubcoreMesh(core_axis_name="c", subcore_axis_name="s")


def _deg_body(edges, degp, degacc, dstb, onesb):
    c = lax.axis_index("c")
    t = lax.axis_index("s")

    def fill_z(r, carry):
        for j in range(FH // 16):
            onesb[r, pl.ds(j * 16, 16)] = jnp.zeros((16,), jnp.float32)
        return carry

    def fill_1(r, carry):
        for j in range(FH // 16):
            onesb[r, pl.ds(j * 16, 16)] = jnp.ones((16,), jnp.float32)
        return carry

    lax.fori_loop(0, CH, fill_z, 0)

    def zc(k, carry):
        pltpu.sync_copy(onesb, degacc.at[pl.ds(t * RPT + k * CH, CH)])
        return carry

    lax.fori_loop(0, RPT // CH, zc, 0)
    lax.fori_loop(0, CH, fill_1, 0)
    plsc.subcore_barrier()

    n0 = (t * NCHUNK_H) // NS
    n1 = ((t + 1) * NCHUNK_H) // NS

    def chunk(n, carry):
        e0 = c * (E // 2) + n * CH
        pltpu.sync_copy(edges.at[1, pl.ds(e0, CH)], dstb)
        pltpu.sync_copy(onesb, degacc.at[dstb], add=True)
        return carry

    lax.fori_loop(n0, n1, chunk, 0)
    plsc.subcore_barrier()
    pltpu.sync_copy(degacc.at[pl.ds(t * RPT, RPT)],
                    degp.at[pl.ds(c * NP + t * RPT, RPT)])


_DEG_SCRATCH = [
    pltpu.VMEM_SHARED((NP, FH), jnp.float32),
    pltpu.VMEM((CH,), jnp.int32),
    pltpu.VMEM((CH, FH), jnp.float32),
]
_deg_call = pl.kernel(
    _deg_body,
    out_type=jax.ShapeDtypeStruct((NC * NP, FH), jnp.float32),
    mesh=_MESH,
    scratch_types=_DEG_SCRATCH,
)


NCHUNK_PAD = 1280
CPT = NCHUNK_PAD // NS
TOT_E = NCHUNK_PAD * CH
NPA = 10112
RPA = NPA // NS


def _hop_body(gtab, tbl, out, acc, r0, r1,
              sa0, sa1, sa2, sa3, da0, da1, da2, da3,
              sg0, sg1, ss0, ss1, si0, si1, si2, si3):
    c = lax.axis_index("c")
    t = lax.axis_index("s")
    rows = [r0, r1]
    sa = [sa0, sa1, sa2, sa3]
    da = [da0, da1, da2, da3]
    sg = [sg0, sg1]
    ss = [ss0, ss1]
    si = [si0, si1, si2, si3]

    def zr(r, carry):
        for j in range(FH // 16):
            r0[r, pl.ds(j * 16, 16)] = jnp.zeros((16,), jnp.float32)
        return carry

    lax.fori_loop(0, CH, zr, 0)

    def zc(k, carry):
        pltpu.sync_copy(r0, acc.at[pl.ds(t * RPA + k * CH, CH)])
        return carry

    lax.fori_loop(0, RPA // CH, zc, 0)
    pltpu.sync_copy(r0.at[pl.ds(0, RPA % CH)],
                    acc.at[pl.ds(t * RPA + (RPA // CH) * CH, RPA % CH)])

    sbase = c * TOT_E + t * (CPT * CH)
    dbase = 2 * TOT_E + t * (CPT * CH)

    def ld_idx(n, p):
        pltpu.async_copy(tbl.at[pl.ds(sbase + n * CH, CH)], sa[p], si[p])
        pltpu.async_copy(tbl.at[pl.ds(dbase + n * CH, CH)], da[p], si[p])

    def wait_idx(n, p):
        pltpu.make_async_copy(tbl.at[pl.ds(sbase + n * CH, CH)],
                              sa[p], si[p]).wait()
        pltpu.make_async_copy(tbl.at[pl.ds(dbase + n * CH, CH)],
                              da[p], si[p]).wait()

    plsc.subcore_barrier()

    ld_idx(0, 0)
    ld_idx(1, 1)
    wait_idx(0, 0)
    pltpu.async_copy(gtab.at[sa[0]], rows[0], sg[0])

    def quad(k, carry):
        for q in range(4):
            n = 4 * k + q
            b = q % 2
            p, pn, p2 = q, (q + 1) % 4, (q + 2) % 4
            pltpu.make_async_copy(gtab.at[sa[p]], rows[b], sg[b]).wait()
            pltpu.async_copy(rows[b], acc.at[da[p]], ss[b], add=True)

            @pl.when(n < CPT - 1)
            def _():
                wait_idx(n + 1, pn)

            @pl.when(n >= 1)
            def _():
                pltpu.make_async_copy(rows[1 - b], acc.at[da[(q + 3) % 4]],
                                      ss[1 - b]).wait()

            @pl.when(n < CPT - 1)
            def _():
                pltpu.async_copy(gtab.at[sa[pn]], rows[1 - b], sg[1 - b])

            @pl.when(n < CPT - 2)
            def _():
                ld_idx(n + 2, p2)

        return carry

    lax.fori_loop(0, CPT // 4, quad, 0)
    pltpu.make_async_copy(rows[1], acc.at[da[3]], ss[1]).wait()

    plsc.subcore_barrier()
    pltpu.sync_copy(acc.at[pl.ds(t * RPA, RPA)],
                    out.at[pl.ds(c * NP + t * RPA, RPA)])


_HOP_SCRATCH = [
    pltpu.VMEM_SHARED((NPA, FH), jnp.float32),
    pltpu.VMEM((CH, FH), jnp.float32),
    pltpu.VMEM((CH, FH), jnp.float32),
] + [pltpu.VMEM((CH,), jnp.int32)] * 8 + [pltpu.SemaphoreType.DMA] * 8
_hop_call = pl.kernel(
    _hop_body,
    out_type=jax.ShapeDtypeStruct((NC * NP, FH), jnp.float32),
    mesh=_MESH,
    scratch_types=_HOP_SCRATCH,
)


def _prep_body(x_ref, degp_ref, nrm_ref, nrm2_ref, g0_ref):
    d = jnp.maximum(degp_ref[0][:, :16] + degp_ref[1][:, :16], 1.0)
    nr = lax.rsqrt(d)
    nrm_ref[...] = nr
    nrm2_ref[...] = nr * nr
    s = nr[:, :1]
    g0_ref[0] = x_ref[:, :FH] * s
    g0_ref[1] = x_ref[:, FH:] * s


def _prep_call(x, degp):
    bn = 640
    return pl.pallas_call(
        _prep_body,
        grid=(NP // bn,),
        in_specs=[
            pl.BlockSpec((bn, F), lambda i: (i, 0)),
            pl.BlockSpec((NC, bn, FH), lambda i: (0, i, 0)),
        ],
        out_specs=[
            pl.BlockSpec((bn, 16), lambda i: (i, 0)),
            pl.BlockSpec((bn, 16), lambda i: (i, 0)),
            pl.BlockSpec((NC, bn, FH), lambda i: (0, i, 0)),
        ],
        out_shape=[
            jax.ShapeDtypeStruct((NP, 16), jnp.float32),
            jax.ShapeDtypeStruct((NP, 16), jnp.float32),
            jax.ShapeDtypeStruct((NC, NP, FH), jnp.float32),
        ],
    )(x, degp)


def _scale_body(s_ref, nrm2_ref, g_ref):
    s2 = nrm2_ref[:, :1]
    g_ref[0] = s_ref[0] * s2
    g_ref[1] = s_ref[1] * s2


def _scale_call(s, nrm2):
    bn = 640
    return pl.pallas_call(
        _scale_body,
        grid=(NP // bn,),
        in_specs=[
            pl.BlockSpec((NC, bn, FH), lambda i: (0, i, 0)),
            pl.BlockSpec((bn, 16), lambda i: (i, 0)),
        ],
        out_specs=pl.BlockSpec((NC, bn, FH), lambda i: (0, i, 0)),
        out_shape=jax.ShapeDtypeStruct((NC, NP, FH), jnp.float32),
    )(s, nrm2)


def _head_body(s_ref, nrm_ref, wsg_ref, bsg_ref, wh_ref, bh_ref,
               h_ref, log_ref):
    nr = nrm_ref[:, :1]
    p0 = s_ref[0] * nr
    p1 = s_ref[1] * nr
    h = jnp.dot(p0, wsg_ref[:FH, :], preferred_element_type=jnp.float32)
    h = h + jnp.dot(p1, wsg_ref[FH:, :], preferred_element_type=jnp.float32)
    h = h + bsg_ref[...]
    h_ref[...] = h
    log_ref[...] = (
        jnp.dot(h, wh_ref[...], preferred_element_type=jnp.float32)
        + bh_ref[...]
    )


def _head_call(s, nrm, W_sg, b_sg, W_head, b_head):
    bn = 1000
    NH = W_sg.shape[1]
    NCL = W_head.shape[1]
    return pl.pallas_call(
        _head_body,
        grid=(N // bn,),
        in_specs=[
            pl.BlockSpec((NC, bn, FH), lambda i: (0, i, 0)),
            pl.BlockSpec((bn, 16), lambda i: (i, 0)),
            pl.BlockSpec((F, NH), lambda i: (0, 0)),
            pl.BlockSpec((1, NH), lambda i: (0, 0)),
            pl.BlockSpec((NH, NCL), lambda i: (0, 0)),
            pl.BlockSpec((1, NCL), lambda i: (0, 0)),
        ],
        out_specs=[
            pl.BlockSpec((bn, NH), lambda i: (i, 0)),
            pl.BlockSpec((bn, NCL), lambda i: (i, 0)),
        ],
        out_shape=[
            jax.ShapeDtypeStruct((N, NH), jnp.float32),
            jax.ShapeDtypeStruct((N, NCL), jnp.float32),
        ],
    )(s, nrm, W_sg, b_sg, W_head, b_head)


def kernel(features, edge_index, W_sg, b_sg, W_head, b_head):
    pad = TOT_E - E
    pad_idx = jnp.arange(pad, dtype=jnp.int32) % (NPA - N) + N
    src_p = jnp.concatenate([edge_index[0], pad_idx])
    dst_p = jnp.concatenate([edge_index[1], pad_idx])
    tbl = jnp.concatenate([src_p, src_p + NP, dst_p])
    degp = _deg_call(edge_index).reshape(NC, NP, FH)
    nrm, nrm2, g0 = _prep_call(features, degp)
    s1 = _hop_call(g0.reshape(NC * NP, FH), tbl)
    g1 = _scale_call(s1.reshape(NC, NP, FH), nrm2)
    s2 = _hop_call(g1.reshape(NC * NP, FH), tbl)
    h, logits = _head_call(s2.reshape(NC, NP, FH), nrm,
                           W_sg, b_sg.reshape(1, -1),
                           W_head, b_head.reshape(1, -1))
    return (h, logits)


# speedup vs baseline: 6.3627x; 1.0158x over previous
import functools

import jax
import jax.numpy as jnp
from jax import lax
from jax.experimental import pallas as pl
from jax.experimental.pallas import tpu as pltpu
from jax.experimental.pallas import tpu_sc as plsc

N = 10000
NP = 10240
E = 160000
F = 256
FH = 128
NC = 2
NS = 16
RPT = NP // NS
CH = 128
NCHUNK = E // CH
NCHUNK_H = NCHUNK // 2

_MESH = plsc.VectorSubcoreMesh(core_axis_name="c", subcore_axis_name="s")


DCPT = 40


def _deg_body(tbl, degp, degacc, onesb, db0, db1, db2, db3,
              sd0, sd1, sd2, sd3, ssd0, ssd1):
    c = lax.axis_index("c")
    t = lax.axis_index("s")
    db = [db0, db1, db2, db3]
    sd = [sd0, sd1, sd2, sd3]
    ssd = [ssd0, ssd1]

    def fill_z(r, carry):
        for j in range(FH // 16):
            onesb[r, pl.ds(j * 16, 16)] = jnp.zeros((16,), jnp.float32)
        return carry

    def fill_1(r, carry):
        for j in range(FH // 16):
            onesb[r, pl.ds(j * 16, 16)] = jnp.ones((16,), jnp.float32)
        return carry

    lax.fori_loop(0, CH, fill_z, 0)

    def zc(k, carry):
        pltpu.sync_copy(onesb, degacc.at[pl.ds(t * RPT + k * CH, CH)])
        return carry

    lax.fori_loop(0, RPT // CH, zc, 0)
    lax.fori_loop(0, CH, fill_1, 0)

    dbase = 2 * TOT_E + (c * (NCHUNK_PAD // 2) + t * DCPT) * CH

    def ld_idx(n, p):
        pltpu.async_copy(tbl.at[pl.ds(dbase + n * CH, CH)], db[p], sd[p])

    def wait_idx(n, p):
        pltpu.make_async_copy(tbl.at[pl.ds(dbase + n * CH, CH)],
                              db[p], sd[p]).wait()

    ld_idx(0, 0)
    ld_idx(1, 1)
    plsc.subcore_barrier()

    def quad(k, carry):
        for q in range(4):
            n = 4 * k + q
            b = q % 2

            @pl.when(n >= 2)
            def _():
                pltpu.make_async_copy(onesb, degacc.at[db[(q + 2) % 4]],
                                      ssd[b]).wait()

            @pl.when(n < DCPT - 2)
            def _():
                ld_idx(n + 2, (q + 2) % 4)

            wait_idx(n, q)
            pltpu.async_copy(onesb, degacc.at[db[q]], ssd[b], add=True)

        return carry

    lax.fori_loop(0, DCPT // 4, quad, 0)
    pltpu.make_async_copy(onesb, degacc.at[db[2]], ssd[0]).wait()
    pltpu.make_async_copy(onesb, degacc.at[db[3]], ssd[1]).wait()

    plsc.subcore_barrier()
    pltpu.sync_copy(degacc.at[pl.ds(t * RPT, RPT)],
                    degp.at[pl.ds(c * NP + t * RPT, RPT)])


_DEG_SCRATCH = [
    pltpu.VMEM_SHARED((NP, FH), jnp.float32),
    pltpu.VMEM((CH, FH), jnp.float32),
] + [pltpu.VMEM((CH,), jnp.int32)] * 4 + [pltpu.SemaphoreType.DMA] * 6
_deg_call = pl.kernel(
    _deg_body,
    out_type=jax.ShapeDtypeStruct((NC * NP, FH), jnp.float32),
    mesh=_MESH,
    scratch_types=_DEG_SCRATCH,
)


NCHUNK_PAD = 1280
CPT = NCHUNK_PAD // NS
TOT_E = NCHUNK_PAD * CH
NPA = 10112
RPA = NPA // NS


def _hop_body(gtab, tbl, out, acc, r0, r1,
              sa0, sa1, sa2, sa3, da0, da1, da2, da3,
              sg0, sg1, ss0, ss1, si0, si1, si2, si3):
    c = lax.axis_index("c")
    t = lax.axis_index("s")
    rows = [r0, r1]
    sa = [sa0, sa1, sa2, sa3]
    da = [da0, da1, da2, da3]
    sg = [sg0, sg1]
    ss = [ss0, ss1]
    si = [si0, si1, si2, si3]

    def zr(r, carry):
        for j in range(FH // 16):
            r0[r, pl.ds(j * 16, 16)] = jnp.zeros((16,), jnp.float32)
        return carry

    lax.fori_loop(0, CH, zr, 0)

    def zc(k, carry):
        pltpu.sync_copy(r0, acc.at[pl.ds(t * RPA + k * CH, CH)])
        return carry

    lax.fori_loop(0, RPA // CH, zc, 0)
    pltpu.sync_copy(r0.at[pl.ds(0, RPA % CH)],
                    acc.at[pl.ds(t * RPA + (RPA // CH) * CH, RPA % CH)])

    sbase = c * TOT_E + t * (CPT * CH)
    dbase = 2 * TOT_E + t * (CPT * CH)

    def ld_idx(n, p):
        pltpu.async_copy(tbl.at[pl.ds(sbase + n * CH, CH)], sa[p], si[p])
        pltpu.async_copy(tbl.at[pl.ds(dbase + n * CH, CH)], da[p], si[p])

    def wait_idx(n, p):
        pltpu.make_async_copy(tbl.at[pl.ds(sbase + n * CH, CH)],
                              sa[p], si[p]).wait()
        pltpu.make_async_copy(tbl.at[pl.ds(dbase + n * CH, CH)],
                              da[p], si[p]).wait()

    plsc.subcore_barrier()

    ld_idx(0, 0)
    ld_idx(1, 1)
    wait_idx(0, 0)
    pltpu.async_copy(gtab.at[sa[0]], rows[0], sg[0])

    def quad(k, carry):
        for q in range(4):
            n = 4 * k + q
            b = q % 2
            p, pn, p2 = q, (q + 1) % 4, (q + 2) % 4
            pltpu.make_async_copy(gtab.at[sa[p]], rows[b], sg[b]).wait()
            pltpu.async_copy(rows[b], acc.at[da[p]], ss[b], add=True)

            @pl.when(n < CPT - 1)
            def _():
                wait_idx(n + 1, pn)

            @pl.when(n >= 1)
            def _():
                pltpu.make_async_copy(rows[1 - b], acc.at[da[(q + 3) % 4]],
                                      ss[1 - b]).wait()

            @pl.when(n < CPT - 1)
            def _():
                pltpu.async_copy(gtab.at[sa[pn]], rows[1 - b], sg[1 - b])

            @pl.when(n < CPT - 2)
            def _():
                ld_idx(n + 2, p2)

        return carry

    lax.fori_loop(0, CPT // 4, quad, 0)
    pltpu.make_async_copy(rows[1], acc.at[da[3]], ss[1]).wait()

    plsc.subcore_barrier()
    pltpu.sync_copy(acc.at[pl.ds(t * RPA, RPA)],
                    out.at[pl.ds(c * NP + t * RPA, RPA)])


_HOP_SCRATCH = [
    pltpu.VMEM_SHARED((NPA, FH), jnp.float32),
    pltpu.VMEM((CH, FH), jnp.float32),
    pltpu.VMEM((CH, FH), jnp.float32),
] + [pltpu.VMEM((CH,), jnp.int32)] * 8 + [pltpu.SemaphoreType.DMA] * 8
_hop_call = pl.kernel(
    _hop_body,
    out_type=jax.ShapeDtypeStruct((NC * NP, FH), jnp.float32),
    mesh=_MESH,
    scratch_types=_HOP_SCRATCH,
)


def _prep_body(x_ref, degp_ref, nrm_ref, nrm2_ref, g0_ref):
    d = jnp.maximum(degp_ref[0][:, :16] + degp_ref[1][:, :16], 1.0)
    nr = lax.rsqrt(d)
    nrm_ref[...] = nr
    nrm2_ref[...] = nr * nr
    s = nr[:, :1]
    g0_ref[0] = x_ref[:, :FH] * s
    g0_ref[1] = x_ref[:, FH:] * s


def _prep_call(x, degp):
    bn = 640
    return pl.pallas_call(
        _prep_body,
        grid=(NP // bn,),
        in_specs=[
            pl.BlockSpec((bn, F), lambda i: (i, 0)),
            pl.BlockSpec((NC, bn, FH), lambda i: (0, i, 0)),
        ],
        out_specs=[
            pl.BlockSpec((bn, 16), lambda i: (i, 0)),
            pl.BlockSpec((bn, 16), lambda i: (i, 0)),
            pl.BlockSpec((NC, bn, FH), lambda i: (0, i, 0)),
        ],
        out_shape=[
            jax.ShapeDtypeStruct((NP, 16), jnp.float32),
            jax.ShapeDtypeStruct((NP, 16), jnp.float32),
            jax.ShapeDtypeStruct((NC, NP, FH), jnp.float32),
        ],
    )(x, degp)


def _scale_body(s_ref, nrm2_ref, g_ref):
    s2 = nrm2_ref[:, :1]
    g_ref[0] = s_ref[0] * s2
    g_ref[1] = s_ref[1] * s2


def _scale_call(s, nrm2):
    bn = 640
    return pl.pallas_call(
        _scale_body,
        grid=(NP // bn,),
        in_specs=[
            pl.BlockSpec((NC, bn, FH), lambda i: (0, i, 0)),
            pl.BlockSpec((bn, 16), lambda i: (i, 0)),
        ],
        out_specs=pl.BlockSpec((NC, bn, FH), lambda i: (0, i, 0)),
        out_shape=jax.ShapeDtypeStruct((NC, NP, FH), jnp.float32),
    )(s, nrm2)


def _head_body(s_ref, nrm_ref, wsg_ref, bsg_ref, wh_ref, bh_ref,
               h_ref, log_ref):
    nr = nrm_ref[:, :1]
    bf = jnp.bfloat16
    p0 = (s_ref[0] * nr).astype(bf)
    p1 = (s_ref[1] * nr).astype(bf)
    h = jnp.dot(p0, wsg_ref[:FH, :].astype(bf),
                preferred_element_type=jnp.float32)
    h = h + jnp.dot(p1, wsg_ref[FH:, :].astype(bf),
                    preferred_element_type=jnp.float32)
    h = h + bsg_ref[...]
    h_ref[...] = h
    log_ref[...] = (
        jnp.dot(h.astype(bf), wh_ref[...].astype(bf),
                preferred_element_type=jnp.float32)
        + bh_ref[...]
    )


def _head_call(s, nrm, W_sg, b_sg, W_head, b_head):
    bn = 1000
    NH = W_sg.shape[1]
    NCL = W_head.shape[1]
    return pl.pallas_call(
        _head_body,
        grid=(N // bn,),
        in_specs=[
            pl.BlockSpec((NC, bn, FH), lambda i: (0, i, 0)),
            pl.BlockSpec((bn, 16), lambda i: (i, 0)),
            pl.BlockSpec((F, NH), lambda i: (0, 0)),
            pl.BlockSpec((1, NH), lambda i: (0, 0)),
            pl.BlockSpec((NH, NCL), lambda i: (0, 0)),
            pl.BlockSpec((1, NCL), lambda i: (0, 0)),
        ],
        out_specs=[
            pl.BlockSpec((bn, NH), lambda i: (i, 0)),
            pl.BlockSpec((bn, NCL), lambda i: (i, 0)),
        ],
        out_shape=[
            jax.ShapeDtypeStruct((N, NH), jnp.float32),
            jax.ShapeDtypeStruct((N, NCL), jnp.float32),
        ],
    )(s, nrm, W_sg, b_sg, W_head, b_head)


def kernel(features, edge_index, W_sg, b_sg, W_head, b_head):
    pad = TOT_E - E
    pad_idx = jnp.arange(pad, dtype=jnp.int32) % (NPA - N) + N
    src_p = jnp.concatenate([edge_index[0], pad_idx])
    dst_p = jnp.concatenate([edge_index[1], pad_idx])
    tbl = jnp.concatenate([src_p, src_p + NP, dst_p])
    degp = _deg_call(tbl).reshape(NC, NP, FH)
    nrm, nrm2, g0 = _prep_call(features, degp)
    s1 = _hop_call(g0.reshape(NC * NP, FH), tbl)
    g1 = _scale_call(s1.reshape(NC, NP, FH), nrm2)
    s2 = _hop_call(g1.reshape(NC * NP, FH), tbl)
    h, logits = _head_call(s2.reshape(NC, NP, FH), nrm,
                           W_sg, b_sg.reshape(1, -1),
                           W_head, b_head.reshape(1, -1))
    return (h, logits)


# speedup vs baseline: 6.3934x; 1.0048x over previous
import jax
import jax.numpy as jnp
from jax import lax
from jax.experimental import pallas as pl
from jax.experimental.pallas import tpu as pltpu
from jax.experimental.pallas import tpu_sc as plsc

N = 10000
NP = 10240
E = 160000
F = 256
FH = 128
NC = 2
NS = 16
RPT = NP // NS
CH = 128
NCHUNK = E // CH

_MESH = plsc.VectorSubcoreMesh(core_axis_name="c", subcore_axis_name="s")


DCPT = 40


def _deg_body(tbl, degp, degacc, onesb, db0, db1, db2, db3,
              sd0, sd1, sd2, sd3, ssd0, ssd1):
    c = lax.axis_index("c")
    t = lax.axis_index("s")
    db = [db0, db1, db2, db3]
    sd = [sd0, sd1, sd2, sd3]
    ssd = [ssd0, ssd1]

    def fill_z(r, carry):
        for j in range(FH // 16):
            onesb[r, pl.ds(j * 16, 16)] = jnp.zeros((16,), jnp.float32)
        return carry

    def fill_1(r, carry):
        for j in range(FH // 16):
            onesb[r, pl.ds(j * 16, 16)] = jnp.ones((16,), jnp.float32)
        return carry

    lax.fori_loop(0, CH, fill_z, 0)

    def zc(k, carry):
        pltpu.sync_copy(onesb, degacc.at[pl.ds(t * RPT + k * CH, CH)])
        return carry

    lax.fori_loop(0, RPT // CH, zc, 0)
    lax.fori_loop(0, CH, fill_1, 0)

    dbase = 2 * TOT_E + (c * (NCHUNK_PAD // 2) + t * DCPT) * CH

    def ld_idx(n, p):
        pltpu.async_copy(tbl.at[pl.ds(dbase + n * CH, CH)], db[p], sd[p])

    def wait_idx(n, p):
        pltpu.make_async_copy(tbl.at[pl.ds(dbase + n * CH, CH)],
                              db[p], sd[p]).wait()

    ld_idx(0, 0)
    ld_idx(1, 1)
    plsc.subcore_barrier()

    def quad(k, carry):
        for q in range(4):
            n = 4 * k + q
            b = q % 2

            @pl.when(n >= 2)
            def _():
                pltpu.make_async_copy(onesb, degacc.at[db[(q + 2) % 4]],
                                      ssd[b]).wait()

            @pl.when(n < DCPT - 2)
            def _():
                ld_idx(n + 2, (q + 2) % 4)

            wait_idx(n, q)
            pltpu.async_copy(onesb, degacc.at[db[q]], ssd[b], add=True)

        return carry

    lax.fori_loop(0, DCPT // 4, quad, 0)
    pltpu.make_async_copy(onesb, degacc.at[db[2]], ssd[0]).wait()
    pltpu.make_async_copy(onesb, degacc.at[db[3]], ssd[1]).wait()

    plsc.subcore_barrier()
    pltpu.sync_copy(degacc.at[pl.ds(t * RPT, RPT)],
                    degp.at[pl.ds(c * NP + t * RPT, RPT)])


_DEG_SCRATCH = [
    pltpu.VMEM_SHARED((NP, FH), jnp.float32),
    pltpu.VMEM((CH, FH), jnp.float32),
] + [pltpu.VMEM((CH,), jnp.int32)] * 4 + [pltpu.SemaphoreType.DMA] * 6
_deg_call = pl.kernel(
    _deg_body,
    out_type=jax.ShapeDtypeStruct((NC * NP, FH), jnp.float32),
    mesh=_MESH,
    scratch_types=_DEG_SCRATCH,
)


NCHUNK_PAD = 1280
CPT = NCHUNK_PAD // NS
TOT_E = NCHUNK_PAD * CH
NPA = 10112
RPA = NPA // NS


def _hop_body(gtab, tbl, out, acc, r0, r1,
              sa0, sa1, sa2, sa3, da0, da1, da2, da3,
              sg0, sg1, ss0, ss1, si0, si1, si2, si3):
    c = lax.axis_index("c")
    t = lax.axis_index("s")
    rows = [r0, r1]
    sa = [sa0, sa1, sa2, sa3]
    da = [da0, da1, da2, da3]
    sg = [sg0, sg1]
    ss = [ss0, ss1]
    si = [si0, si1, si2, si3]

    def zr(r, carry):
        for j in range(FH // 16):
            r0[r, pl.ds(j * 16, 16)] = jnp.zeros((16,), jnp.float32)
        return carry

    lax.fori_loop(0, CH, zr, 0)

    def zc(k, carry):
        pltpu.sync_copy(r0, acc.at[pl.ds(t * RPA + k * CH, CH)])
        return carry

    lax.fori_loop(0, RPA // CH, zc, 0)
    pltpu.sync_copy(r0.at[pl.ds(0, RPA % CH)],
                    acc.at[pl.ds(t * RPA + (RPA // CH) * CH, RPA % CH)])

    sbase = c * TOT_E + t * (CPT * CH)
    dbase = 2 * TOT_E + t * (CPT * CH)

    def ld_idx(n, p):
        pltpu.async_copy(tbl.at[pl.ds(sbase + n * CH, CH)], sa[p], si[p])
        pltpu.async_copy(tbl.at[pl.ds(dbase + n * CH, CH)], da[p], si[p])

    def wait_idx(n, p):
        pltpu.make_async_copy(tbl.at[pl.ds(sbase + n * CH, CH)],
                              sa[p], si[p]).wait()
        pltpu.make_async_copy(tbl.at[pl.ds(dbase + n * CH, CH)],
                              da[p], si[p]).wait()

    plsc.subcore_barrier()

    ld_idx(0, 0)
    ld_idx(1, 1)
    wait_idx(0, 0)
    pltpu.async_copy(gtab.at[sa[0]], rows[0], sg[0])

    def quad(k, carry):
        for q in range(4):
            n = 4 * k + q
            b = q % 2
            p, pn, p2 = q, (q + 1) % 4, (q + 2) % 4
            pltpu.make_async_copy(gtab.at[sa[p]], rows[b], sg[b]).wait()
            pltpu.async_copy(rows[b], acc.at[da[p]], ss[b], add=True)

            @pl.when(n < CPT - 1)
            def _():
                wait_idx(n + 1, pn)

            @pl.when(n >= 1)
            def _():
                pltpu.make_async_copy(rows[1 - b], acc.at[da[(q + 3) % 4]],
                                      ss[1 - b]).wait()

            @pl.when(n < CPT - 1)
            def _():
                pltpu.async_copy(gtab.at[sa[pn]], rows[1 - b], sg[1 - b])

            @pl.when(n < CPT - 2)
            def _():
                ld_idx(n + 2, p2)

        return carry

    lax.fori_loop(0, CPT // 4, quad, 0)
    pltpu.make_async_copy(rows[1], acc.at[da[3]], ss[1]).wait()

    plsc.subcore_barrier()
    pltpu.sync_copy(acc.at[pl.ds(t * RPA, RPA)],
                    out.at[pl.ds(c * NP + t * RPA, RPA)])


_HOP_SCRATCH = [
    pltpu.VMEM_SHARED((NPA, FH), jnp.float32),
    pltpu.VMEM((CH, FH), jnp.float32),
    pltpu.VMEM((CH, FH), jnp.float32),
] + [pltpu.VMEM((CH,), jnp.int32)] * 8 + [pltpu.SemaphoreType.DMA] * 8
_hop_call = pl.kernel(
    _hop_body,
    out_type=jax.ShapeDtypeStruct((NC * NP, FH), jnp.float32),
    mesh=_MESH,
    scratch_types=_HOP_SCRATCH,
)


def _prep_body(x_ref, degp_ref, nrm_ref, nrm2_ref, g0_ref):
    d = jnp.maximum(degp_ref[0][:, :16] + degp_ref[1][:, :16], 1.0)
    nr = lax.rsqrt(d)
    nrm_ref[...] = nr
    nrm2_ref[...] = nr * nr
    s = nr[:, :1]
    g0_ref[0] = x_ref[:, :FH] * s
    g0_ref[1] = x_ref[:, FH:] * s


def _prep_call(x, degp):
    bn = 640
    return pl.pallas_call(
        _prep_body,
        grid=(NP // bn,),
        in_specs=[
            pl.BlockSpec((bn, F), lambda i: (i, 0)),
            pl.BlockSpec((NC, bn, FH), lambda i: (0, i, 0)),
        ],
        out_specs=[
            pl.BlockSpec((bn, 16), lambda i: (i, 0)),
            pl.BlockSpec((bn, 16), lambda i: (i, 0)),
            pl.BlockSpec((NC, bn, FH), lambda i: (0, i, 0)),
        ],
        out_shape=[
            jax.ShapeDtypeStruct((NP, 16), jnp.float32),
            jax.ShapeDtypeStruct((NP, 16), jnp.float32),
            jax.ShapeDtypeStruct((NC, NP, FH), jnp.float32),
        ],
    )(x, degp)


def _scale_body(s_ref, nrm2_ref, g_ref):
    s2 = nrm2_ref[:, :1]
    g_ref[0] = s_ref[0] * s2
    g_ref[1] = s_ref[1] * s2


def _scale_call(s, nrm2):
    bn = 640
    return pl.pallas_call(
        _scale_body,
        grid=(NP // bn,),
        in_specs=[
            pl.BlockSpec((NC, bn, FH), lambda i: (0, i, 0)),
            pl.BlockSpec((bn, 16), lambda i: (i, 0)),
        ],
        out_specs=pl.BlockSpec((NC, bn, FH), lambda i: (0, i, 0)),
        out_shape=jax.ShapeDtypeStruct((NC, NP, FH), jnp.float32),
    )(s, nrm2)


def _head_body(s_ref, nrm_ref, wsg_ref, bsg_ref, wh_ref, bh_ref,
               h_ref, log_ref):
    nr = nrm_ref[:, :1]
    bf = jnp.bfloat16
    p0 = (s_ref[0] * nr).astype(bf)
    p1 = (s_ref[1] * nr).astype(bf)
    h = jnp.dot(p0, wsg_ref[:FH, :].astype(bf),
                preferred_element_type=jnp.float32)
    h = h + jnp.dot(p1, wsg_ref[FH:, :].astype(bf),
                    preferred_element_type=jnp.float32)
    h = h + bsg_ref[...]
    h_ref[...] = h
    log_ref[...] = (
        jnp.dot(h.astype(bf), wh_ref[...].astype(bf),
                preferred_element_type=jnp.float32)
        + bh_ref[...]
    )


def _head_call(s, nrm, W_sg, b_sg, W_head, b_head):
    bn = 1000
    NH = W_sg.shape[1]
    NCL = W_head.shape[1]
    return pl.pallas_call(
        _head_body,
        grid=(N // bn,),
        in_specs=[
            pl.BlockSpec((NC, bn, FH), lambda i: (0, i, 0)),
            pl.BlockSpec((bn, 16), lambda i: (i, 0)),
            pl.BlockSpec((F, NH), lambda i: (0, 0)),
            pl.BlockSpec((1, NH), lambda i: (0, 0)),
            pl.BlockSpec((NH, NCL), lambda i: (0, 0)),
            pl.BlockSpec((1, NCL), lambda i: (0, 0)),
        ],
        out_specs=[
            pl.BlockSpec((bn, NH), lambda i: (i, 0)),
            pl.BlockSpec((bn, NCL), lambda i: (i, 0)),
        ],
        out_shape=[
            jax.ShapeDtypeStruct((N, NH), jnp.float32),
            jax.ShapeDtypeStruct((N, NCL), jnp.float32),
        ],
    )(s, nrm, W_sg, b_sg, W_head, b_head)


def kernel(features, edge_index, W_sg, b_sg, W_head, b_head):
    pad = TOT_E - E
    pad_idx = jnp.arange(pad, dtype=jnp.int32) % (NPA - N) + N
    src_p = jnp.concatenate([edge_index[0], pad_idx])
    dst_p = jnp.concatenate([edge_index[1], pad_idx])
    tbl = jnp.concatenate([src_p, src_p + NP, dst_p])
    degp = _deg_call(tbl).reshape(NC, NP, FH)
    nrm, nrm2, g0 = _prep_call(features, degp)
    s1 = _hop_call(g0.reshape(NC * NP, FH), tbl)
    g1 = _scale_call(s1.reshape(NC, NP, FH), nrm2)
    s2 = _hop_call(g1.reshape(NC * NP, FH), tbl)
    h, logits = _head_call(s2.reshape(NC, NP, FH), nrm,
                           W_sg, b_sg.reshape(1, -1),
                           W_head, b_head.reshape(1, -1))
    return (h, logits)
